# Initial kernel scaffold; baseline (speedup 1.0000x reference)
import jax
import jax.numpy as jnp
from jax.experimental import pallas as pl


def kernel(x_semantic, x_collaborate, sem_enc_W0, sem_enc_b0, sem_enc_W1, sem_enc_b1, sem_enc_W2, sem_enc_b2, sem_dec_W0, sem_dec_b0, sem_dec_W1, sem_dec_b1, sem_dec_W2, sem_dec_b2, col_enc_W0, col_enc_b0, col_enc_W1, col_enc_b1, col_enc_W2, col_enc_b2, col_dec_W0, col_dec_b0, col_dec_W1, col_dec_b1, col_dec_W2, col_dec_b2, cb_shared_0, cb_sem_0, cb_sem_1, cb_col_0, cb_col_1):
    raise NotImplementedError("write your pallas kernel here")


# speedup vs baseline: 1.4728x; 1.4728x over previous
import jax
import jax.numpy as jnp
from jax import lax
from jax.experimental import pallas as pl
from jax.experimental.pallas import tpu as pltpu

B = 4096
TB = 256
H = 128
PREC = lax.Precision.DEFAULT
HI = lax.Precision.HIGHEST


def _mm(a, b, prec=PREC):
    return jnp.dot(a, b, precision=prec, preferred_element_type=jnp.float32)


def _quant(zres, cb_t, cb):
    rs = jnp.sum(zres * zres, axis=1, keepdims=True)
    cs = jnp.sum(cb * cb, axis=1)[None, :]
    d = rs + cs - 2.0 * _mm(zres, cb_t)
    m = jnp.min(d, axis=1, keepdims=True)
    ii = lax.broadcasted_iota(jnp.int32, d.shape, 1)
    idx = jnp.min(jnp.where(d == m, ii, d.shape[1]), axis=1, keepdims=True)
    oh = (ii == idx).astype(jnp.float32)
    zq = _mm(oh, cb, HI)
    return zq, idx


def _enc_kernel(xs_ref, xc_ref,
                sw0, sb0, sw1, sb1, sw2, sb2,
                cw0, cb0, cw1, cb1, cw2, cb2,
                cbs_t, cbm0_t, cbm1_t, cbc0_t, cbc1_t,
                cbs, cbm0, cbm1, cbc0, cbc1,
                semq_ref, colq_ref, idx_ref, loss_ref):
    i = pl.program_id(0)
    h = jnp.maximum(_mm(xs_ref[...], sw0[...]) + sb0[...], 0.0)
    h = jnp.maximum(_mm(h, sw1[...]) + sb1[...], 0.0)
    zs = _mm(h, sw2[...]) + sb2[...]
    h = jnp.maximum(_mm(xc_ref[...], cw0[...]) + cb0[...], 0.0)
    h = jnp.maximum(_mm(h, cw1[...]) + cb1[...], 0.0)
    zc = _mm(h, cw2[...]) + cb2[...]
    z = jnp.concatenate([zs, zc], axis=1)

    zq0, i0 = _quant(z, cbs_t[...], cbs[...])
    s0 = jnp.sum((zq0 - z) ** 2)
    r = z - zq0
    rs_, rc_ = r[:, :H], r[:, H:]
    qs, qc = zq0[:, :H], zq0[:, H:]

    zq1, i1 = _quant(rs_, cbm0_t[...], cbm0[...])
    s1 = jnp.sum((zq1 - rs_) ** 2)
    rs_ = rs_ - zq1
    qs = qs + zq1
    zq2, i2 = _quant(rs_, cbm1_t[...], cbm1[...])
    s2 = jnp.sum((zq2 - rs_) ** 2)
    qs = qs + zq2

    zq3, i3 = _quant(rc_, cbc0_t[...], cbc0[...])
    s3 = jnp.sum((zq3 - rc_) ** 2)
    rc_ = rc_ - zq3
    qc = qc + zq3
    zq4, i4 = _quant(rc_, cbc1_t[...], cbc1[...])
    s4 = jnp.sum((zq4 - rc_) ** 2)
    qc = qc + zq4

    semq_ref[...] = qs
    colq_ref[...] = qc
    idx_ref[...] = jnp.concatenate([i0, i1, i2, i3, i4], axis=1)

    contrib = (1.25 / 5.0) * (s0 / (B * 256.0) + (s1 + s2 + s3 + s4) / (B * 128.0))

    @pl.when(i == 0)
    def _init():
        loss_ref[...] = jnp.zeros_like(loss_ref)

    loss_ref[...] += jnp.full((1, 1), contrib, jnp.float32)


def _dec_kernel(sq_ref, cq_ref,
                sw0, sb0, sw1, sb1, sw2, sb2,
                cw0, cb0, cw1, cb1, cw2, cb2,
                semo_ref, colo_ref):
    h = jnp.maximum(_mm(sq_ref[...], sw0[...]) + sb0[...], 0.0)
    h = jnp.maximum(_mm(h, sw1[...]) + sb1[...], 0.0)
    semo_ref[...] = _mm(h, sw2[...]) + sb2[...]
    h = jnp.maximum(_mm(cq_ref[...], cw0[...]) + cb0[...], 0.0)
    h = jnp.maximum(_mm(h, cw1[...]) + cb1[...], 0.0)
    colo_ref[...] = _mm(h, cw2[...]) + cb2[...]


def _full_spec(shape):
    return pl.BlockSpec(shape, lambda i: (0,) * len(shape))


def kernel(x_semantic, x_collaborate,
           sem_enc_W0, sem_enc_b0, sem_enc_W1, sem_enc_b1, sem_enc_W2, sem_enc_b2,
           sem_dec_W0, sem_dec_b0, sem_dec_W1, sem_dec_b1, sem_dec_W2, sem_dec_b2,
           col_enc_W0, col_enc_b0, col_enc_W1, col_enc_b1, col_enc_W2, col_enc_b2,
           col_dec_W0, col_dec_b0, col_dec_W1, col_dec_b1, col_dec_W2, col_dec_b2,
           cb_shared_0, cb_sem_0, cb_sem_1, cb_col_0, cb_col_1):
    grid = (B // TB,)
    b2 = lambda b: b.reshape(1, -1)

    enc_ins = [
        sem_enc_W0, b2(sem_enc_b0), sem_enc_W1, b2(sem_enc_b1), sem_enc_W2, b2(sem_enc_b2),
        col_enc_W0, b2(col_enc_b0), col_enc_W1, b2(col_enc_b1), col_enc_W2, b2(col_enc_b2),
        cb_shared_0.T, cb_sem_0.T, cb_sem_1.T, cb_col_0.T, cb_col_1.T,
        cb_shared_0, cb_sem_0, cb_sem_1, cb_col_0, cb_col_1,
    ]
    enc_specs = [_full_spec(a.shape) for a in enc_ins]

    semq, colq, indices, loss = pl.pallas_call(
        _enc_kernel,
        grid=grid,
        in_specs=[
            pl.BlockSpec((TB, 768), lambda i: (i, 0)),
            pl.BlockSpec((TB, 768), lambda i: (i, 0)),
        ] + enc_specs,
        out_specs=[
            pl.BlockSpec((TB, H), lambda i: (i, 0)),
            pl.BlockSpec((TB, H), lambda i: (i, 0)),
            pl.BlockSpec((TB, 5), lambda i: (i, 0)),
            pl.BlockSpec((1, 1), lambda i: (0, 0)),
        ],
        out_shape=[
            jax.ShapeDtypeStruct((B, H), jnp.float32),
            jax.ShapeDtypeStruct((B, H), jnp.float32),
            jax.ShapeDtypeStruct((B, 5), jnp.int32),
            jax.ShapeDtypeStruct((1, 1), jnp.float32),
        ],
        compiler_params=pltpu.CompilerParams(
            dimension_semantics=("arbitrary",),
        ),
    )(x_semantic, x_collaborate, *enc_ins)

    dec_ins = [
        sem_dec_W0, b2(sem_dec_b0), sem_dec_W1, b2(sem_dec_b1), sem_dec_W2, b2(sem_dec_b2),
        col_dec_W0, b2(col_dec_b0), col_dec_W1, b2(col_dec_b1), col_dec_W2, b2(col_dec_b2),
    ]
    dec_specs = [_full_spec(a.shape) for a in dec_ins]

    sem_out, col_out = pl.pallas_call(
        _dec_kernel,
        grid=grid,
        in_specs=[
            pl.BlockSpec((TB, H), lambda i: (i, 0)),
            pl.BlockSpec((TB, H), lambda i: (i, 0)),
        ] + dec_specs,
        out_specs=[
            pl.BlockSpec((TB, 768), lambda i: (i, 0)),
            pl.BlockSpec((TB, 768), lambda i: (i, 0)),
        ],
        out_shape=[
            jax.ShapeDtypeStruct((B, 768), jnp.float32),
            jax.ShapeDtypeStruct((B, 768), jnp.float32),
        ],
        compiler_params=pltpu.CompilerParams(
            dimension_semantics=("arbitrary",),
        ),
    )(semq, colq, *dec_ins)

    return sem_out, col_out, loss.reshape(()), indices
